# Initial kernel scaffold; baseline (speedup 1.0000x reference)
import jax
import jax.numpy as jnp
from jax.experimental import pallas as pl


def kernel(data, params):
    raise NotImplementedError("write your pallas kernel here")


# speedup vs baseline: 4.0734x; 4.0734x over previous
import functools
import math

import jax
import jax.numpy as jnp
import numpy as np
from jax.experimental import pallas as pl

_NEG = np.float32(-1e30)
_F32MAX = np.float32(3e38)


def _cdiv(a, b):
    return (a + b - 1) // b


def _fps_kernel(p_ref, py_ref, *, n_s, np_):
    p = p_ref[0]
    iota = jax.lax.broadcasted_iota(jnp.int32, (1, np_), 1)
    iota_s = jax.lax.broadcasted_iota(jnp.int32, (1, n_s), 1)

    def body(i, st):
        dist, cur, py = st
        m = (iota == cur).astype(jnp.float32)
        pc = jnp.sum(p * m, axis=1, keepdims=True)
        py = jnp.where(iota_s == i, pc, py)
        d = (p[0:1] - pc[0:1]) ** 2 + (p[1:2] - pc[1:2]) ** 2 \
            + (p[2:3] - pc[2:3]) ** 2
        dist = jnp.minimum(dist, d)
        mx = jnp.max(dist)
        cur = jnp.min(jnp.where(dist == mx, iota, np_))
        return dist, cur, py

    dist0 = jnp.full((1, np_), _F32MAX, dtype=jnp.float32)
    py0 = jnp.zeros((3, n_s), dtype=jnp.float32)
    _, _, py = jax.lax.fori_loop(0, n_s, body, (dist0, jnp.int32(0), py0))
    py_ref[0] = py


def _fps_call(p_t, n_s):
    b, _, np_ = p_t.shape
    return pl.pallas_call(
        functools.partial(_fps_kernel, n_s=n_s, np_=np_),
        grid=(b,),
        in_specs=[pl.BlockSpec((1, 3, np_), lambda i: (i, 0, 0))],
        out_specs=pl.BlockSpec((1, 3, n_s), lambda i: (i, 0, 0)),
        out_shape=jax.ShapeDtypeStruct((b, 3, n_s), jnp.float32),
    )(p_t)


def _sa_kernel(p_ref, py_ref, xc_ref, *refs, np_, k, r2, cdim, t, nw):
    ws = refs[:nw]
    out_ref = refs[nw]
    p = p_ref[0]
    py = py_ref[0]
    xc = xc_ref[0]
    iota = jax.lax.broadcasted_iota(jnp.int32, (1, np_), 1)

    d2 = (py[:, 0:1] - p[0:1, :]) ** 2
    d2 = d2 + (py[:, 1:2] - p[1:2, :]) ** 2
    d2 = d2 + (py[:, 2:3] - p[2:3, :]) ** 2
    order = jnp.where(d2 <= r2, iota, np_)

    cout = ws[nw - 2].shape[1]

    def body(_, st):
        order, acc = st
        mk = jnp.min(order, axis=1, keepdims=True)
        order = jnp.where(order == mk, np_ + 1, order)
        valid = mk < np_
        oh = (iota == mk).astype(jnp.float32)
        gj = jnp.dot(oh, xc, preferred_element_type=jnp.float32)
        dp = gj[:, cdim - 3:] - py
        h = jnp.concatenate([gj[:, : cdim - 3], dp], axis=1)
        for li in range(0, nw, 2):
            h = jax.nn.relu(
                jnp.dot(h, ws[li][...], preferred_element_type=jnp.float32)
                + ws[li + 1][...]
            )
        acc = jnp.maximum(acc, jnp.where(valid, h, _NEG))
        return order, acc

    acc0 = jnp.full((t, cout), _NEG, dtype=jnp.float32)
    _, acc = jax.lax.fori_loop(0, k, body, (order, acc0))
    out_ref[0] = acc


def _sa_call(p_t, py_n3, x, mlp, r, k, t=128):
    b, _, np_ = p_t.shape
    n_s = py_n3.shape[1]
    cdim = x.shape[-1] + 3
    xc = jnp.concatenate([x, jnp.transpose(p_t, (0, 2, 1))], axis=-1)
    n_t = _cdiv(n_s, t)
    pad = n_t * t - n_s
    py_pad = jnp.pad(py_n3, ((0, 0), (0, pad), (0, 0)),
                     constant_values=1e9)
    wlist = []
    wspecs = []
    for w_, b_ in mlp:
        wlist += [w_, b_.reshape(1, -1)]
        wspecs += [
            pl.BlockSpec(w_.shape, lambda bi, ti: (0, 0)),
            pl.BlockSpec((1, b_.shape[0]), lambda bi, ti: (0, 0)),
        ]
    cout = mlp[-1][0].shape[1]
    r2 = np.float32(r * r)
    out = pl.pallas_call(
        functools.partial(_sa_kernel, np_=np_, k=k, r2=r2, cdim=cdim,
                          t=t, nw=len(wlist)),
        grid=(b, n_t),
        in_specs=[
            pl.BlockSpec((1, 3, np_), lambda bi, ti: (bi, 0, 0)),
            pl.BlockSpec((1, t, 3), lambda bi, ti: (bi, ti, 0)),
            pl.BlockSpec((1, np_, cdim), lambda bi, ti: (bi, 0, 0)),
        ] + wspecs,
        out_specs=pl.BlockSpec((1, t, cout), lambda bi, ti: (bi, ti, 0)),
        out_shape=jax.ShapeDtypeStruct((b, n_t * t, cout), jnp.float32),
    )(p_t, py_pad, xc, *wlist)
    return out[:, :n_s]


def _fp_kernel(pin_ref, ix_ref, psk_ref, sx_ref, *refs, nin, k, t, nw):
    ws = refs[:nw]
    out_ref = refs[nw]
    pin = pin_ref[0]
    ix = ix_ref[0]
    psk = psk_ref[0]
    sx = sx_ref[0]
    iota = jax.lax.broadcasted_iota(jnp.int32, (1, nin), 1)

    d2 = (psk[:, 0:1] - pin[0:1, :]) ** 2
    d2 = d2 + (psk[:, 1:2] - pin[1:2, :]) ** 2
    d2 = d2 + (psk[:, 2:3] - pin[2:3, :]) ** 2

    idxs = []
    dists = []
    for _ in range(k):
        m = jnp.min(d2, axis=1, keepdims=True)
        idx = jnp.min(jnp.where(d2 == m, iota, nin), axis=1, keepdims=True)
        d2 = jnp.where(iota == idx, _F32MAX, d2)
        idxs.append(idx)
        dists.append(jnp.sqrt(m))

    wsum = jnp.zeros_like(dists[0])
    wts = []
    for d_ in dists:
        w_ = 1.0 / jnp.maximum(d_, np.float32(1e-10))
        wts.append(w_)
        wsum = wsum + w_
    wsum = wsum + np.float32(1e-16)

    a = jnp.zeros((psk.shape[0], nin), dtype=jnp.float32)
    for idx, w_ in zip(idxs, wts):
        a = a + jnp.where(iota == idx, w_ / wsum, 0.0)
    agg = jnp.dot(a, ix, preferred_element_type=jnp.float32)

    h = jnp.concatenate([agg, sx, psk], axis=1)
    for li in range(0, nw, 2):
        h = jax.nn.relu(
            jnp.dot(h, ws[li][...], preferred_element_type=jnp.float32)
            + ws[li + 1][...]
        )
    out_ref[0] = h


def _fp_call(pin_t, ix, psk_n3, sx, mlp, k, t=128):
    b, _, nin = pin_t.shape
    nsk = psk_n3.shape[1]
    c = ix.shape[-1]
    cs = sx.shape[-1]
    n_t = _cdiv(nsk, t)
    pad = n_t * t - nsk
    psk_pad = jnp.pad(psk_n3, ((0, 0), (0, pad), (0, 0)),
                      constant_values=1e9)
    sx_pad = jnp.pad(sx, ((0, 0), (0, pad), (0, 0)))
    wlist = []
    wspecs = []
    for w_, b_ in mlp:
        wlist += [w_, b_.reshape(1, -1)]
        wspecs += [
            pl.BlockSpec(w_.shape, lambda bi, ti: (0, 0)),
            pl.BlockSpec((1, b_.shape[0]), lambda bi, ti: (0, 0)),
        ]
    cout = mlp[-1][0].shape[1]
    out = pl.pallas_call(
        functools.partial(_fp_kernel, nin=nin, k=k, t=t, nw=len(wlist)),
        grid=(b, n_t),
        in_specs=[
            pl.BlockSpec((1, 3, nin), lambda bi, ti: (bi, 0, 0)),
            pl.BlockSpec((1, nin, c), lambda bi, ti: (bi, 0, 0)),
            pl.BlockSpec((1, t, 3), lambda bi, ti: (bi, ti, 0)),
            pl.BlockSpec((1, t, cs), lambda bi, ti: (bi, ti, 0)),
        ] + wspecs,
        out_specs=pl.BlockSpec((1, t, cout), lambda bi, ti: (bi, ti, 0)),
        out_shape=jax.ShapeDtypeStruct((b, n_t * t, cout), jnp.float32),
    )(pin_t, ix, psk_pad, sx_pad, *wlist)
    return out[:, :nsk]


def _clf_kernel(f_ref, *refs, nl, t):
    f = f_ref[0]
    nw = [nl, nl + 1]
    o1_ref, o2_ref = refs[-2], refs[-1]
    offs = 0
    outs = []
    for hd in range(2):
        h = f
        for li in range(nl):
            w_ = refs[offs + li][...]
            h = jnp.dot(h, w_, preferred_element_type=jnp.float32)
            if li == nl - 1:
                h = h + refs[offs + nl][...]
            else:
                h = jax.nn.relu(h)
        outs.append(h)
        offs += nl + 1
    o1_ref[0] = outs[0]
    o2_ref[0] = outs[1]


def _clf_call(f1, clfs, t=512):
    b, n, c = f1.shape
    wlist = []
    wspecs = []
    nl = len(clfs[0])
    for clf in clfs:
        for layer in clf:
            wt = jnp.transpose(layer[0])
            wlist.append(wt)
            wspecs.append(pl.BlockSpec(wt.shape, lambda bi, ti: (0, 0)))
        bias = clf[-1][1].reshape(1, -1)
        wlist.append(bias)
        wspecs.append(pl.BlockSpec(bias.shape, lambda bi, ti: (0, 0)))
    ncls = clfs[0][-1][0].shape[0]
    n_t = _cdiv(n, t)
    o1, o2 = pl.pallas_call(
        functools.partial(_clf_kernel, nl=nl, t=t),
        grid=(b, n_t),
        in_specs=[pl.BlockSpec((1, t, c), lambda bi, ti: (bi, ti, 0))]
        + wspecs,
        out_specs=[
            pl.BlockSpec((1, t, ncls), lambda bi, ti: (bi, ti, 0)),
            pl.BlockSpec((1, t, ncls), lambda bi, ti: (bi, ti, 0)),
        ],
        out_shape=[
            jax.ShapeDtypeStruct((b, n, ncls), jnp.float32),
            jax.ShapeDtypeStruct((b, n, ncls), jnp.float32),
        ],
    )(f1, *wlist)
    return o1, o2


def kernel(data, params):
    pos = jnp.transpose(data, (0, 2, 1))
    pos_t = data
    x = pos

    sa_cfg = [
        ("sa1", 0.8, 0.025, 64),
        ("sa2", 0.7, 0.05, 64),
        ("sa3", 0.6, 0.1, 64),
        ("sa4", 0.5, 0.2, 64),
    ]
    p_t = pos_t
    p_n3 = pos
    xs = [x]
    ps_t = [p_t]
    ps_n3 = [p_n3]
    cur_x = x
    for name, ratio, r, k in sa_cfg:
        np_ = p_t.shape[2]
        n_s = int(math.ceil(ratio * np_))
        py_t = _fps_call(p_t, n_s)
        py_n3 = jnp.transpose(py_t, (0, 2, 1))
        cur_x = _sa_call(p_t, py_n3, cur_x, params[name], r, k)
        p_t, p_n3 = py_t, py_n3
        xs.append(cur_x)
        ps_t.append(p_t)
        ps_n3.append(p_n3)

    f = xs[4]
    fp_cfg = [("fp4", 1, 3), ("fp3", 1, 2), ("fp2", 3, 1), ("fp1", 3, 0)]
    for name, k, si in fp_cfg:
        f = _fp_call(ps_t[si + 1], f, ps_n3[si], xs[si], params[name], k)

    o1, o2 = _clf_call(f, params["classifiers"])
    return (jnp.transpose(o1, (0, 2, 1)), jnp.transpose(o2, (0, 2, 1)))


# speedup vs baseline: 7.7712x; 1.9078x over previous
import functools
import math

import jax
import jax.numpy as jnp
import numpy as np
from jax.experimental import pallas as pl

_NEG = np.float32(-1e30)
_F32MAX = np.float32(3e38)


def _cdiv(a, b):
    return (a + b - 1) // b


def _fps_kernel(p_ref, py_ref, *, n_s, npp):
    s = npp // 128
    p = p_ref[0].reshape(3, s, 128)
    il = (jax.lax.broadcasted_iota(jnp.int32, (s, 128), 0) * 128
          + jax.lax.broadcasted_iota(jnp.int32, (s, 128), 1))
    ic = jax.lax.broadcasted_iota(jnp.int32, (1, 128), 1)

    def step(j, st):
        dist, cur, cx, cy, cz = st
        m = (il == cur).astype(jnp.float32)
        px = jnp.sum(p[0] * m)
        py_ = jnp.sum(p[1] * m)
        pz = jnp.sum(p[2] * m)
        cx = jnp.where(ic == j, px, cx)
        cy = jnp.where(ic == j, py_, cy)
        cz = jnp.where(ic == j, pz, cz)
        d = (p[0] - px) ** 2 + (p[1] - py_) ** 2 + (p[2] - pz) ** 2
        dist = jnp.minimum(dist, d)
        mx = jnp.max(dist)
        cur = jnp.min(jnp.where(dist == mx, il, npp))
        return dist, cur, cx, cy, cz

    dist = jnp.full((s, 128), _F32MAX, dtype=jnp.float32)
    cur = jnp.int32(0)
    z = jnp.zeros((1, 128), dtype=jnp.float32)
    for c0 in range(0, n_s, 128):
        w = min(128, n_s - c0)
        dist, cur, cx, cy, cz = jax.lax.fori_loop(
            0, w, step, (dist, cur, z, z, z))
        py_ref[0, 0:1, c0:c0 + w] = cx[:, :w]
        py_ref[0, 1:2, c0:c0 + w] = cy[:, :w]
        py_ref[0, 2:3, c0:c0 + w] = cz[:, :w]


def _fps_call(p_t, n_s):
    b, _, np_ = p_t.shape
    npp = _cdiv(np_, 128) * 128
    if npp > np_:
        p_t = jnp.concatenate(
            [p_t, jnp.broadcast_to(p_t[:, :, :1], (b, 3, npp - np_))], axis=2)
    return pl.pallas_call(
        functools.partial(_fps_kernel, n_s=n_s, npp=npp),
        grid=(b,),
        in_specs=[pl.BlockSpec((1, 3, npp), lambda i: (i, 0, 0))],
        out_specs=pl.BlockSpec((1, 3, n_s), lambda i: (i, 0, 0)),
        out_shape=jax.ShapeDtypeStruct((b, 3, n_s), jnp.float32),
    )(p_t)


def _sa_kernel(p_ref, py_ref, xc_ref, *refs, np_, k, r2, cdim, t, nw):
    ws = refs[:nw]
    out_ref = refs[nw]
    p = p_ref[0]
    py = py_ref[0]
    xc = xc_ref[0]
    iota = jax.lax.broadcasted_iota(jnp.int32, (1, np_), 1)

    d2 = (py[:, 0:1] - p[0:1, :]) ** 2
    d2 = d2 + (py[:, 1:2] - p[1:2, :]) ** 2
    d2 = d2 + (py[:, 2:3] - p[2:3, :]) ** 2
    valid0 = d2 <= r2
    order = jnp.where(valid0, iota, np_)
    cnt = jnp.sum(valid0.astype(jnp.int32), axis=1, keepdims=True)
    kmax = jnp.minimum(jnp.max(cnt), k)

    cout = ws[nw - 2].shape[1]

    def body(st):
        ki, order, acc = st
        mk = jnp.min(order, axis=1, keepdims=True)
        order = jnp.where(order == mk, np_ + 1, order)
        valid = mk < np_
        oh = (iota == mk).astype(jnp.float32)
        gj = jnp.dot(oh, xc, preferred_element_type=jnp.float32)
        dp = gj[:, cdim - 3:] - py
        h = jnp.concatenate([gj[:, : cdim - 3], dp], axis=1)
        for li in range(0, nw, 2):
            h = jax.nn.relu(
                jnp.dot(h, ws[li][...], preferred_element_type=jnp.float32)
                + ws[li + 1][...]
            )
        acc = jnp.maximum(acc, jnp.where(valid, h, _NEG))
        return ki + 1, order, acc

    acc0 = jnp.full((t, cout), _NEG, dtype=jnp.float32)
    _, _, acc = jax.lax.while_loop(
        lambda st: st[0] < kmax, body, (jnp.int32(0), order, acc0))
    out_ref[0] = acc


def _sa_call(p_t, py_n3, x, mlp, r, k, t=128):
    b, _, np_ = p_t.shape
    n_s = py_n3.shape[1]
    cdim = x.shape[-1] + 3
    xc = jnp.concatenate([x, jnp.transpose(p_t, (0, 2, 1))], axis=-1)
    n_t = _cdiv(n_s, t)
    pad = n_t * t - n_s
    py_pad = jnp.pad(py_n3, ((0, 0), (0, pad), (0, 0)),
                     constant_values=1e9)
    wlist = []
    wspecs = []
    for w_, b_ in mlp:
        wlist += [w_, b_.reshape(1, -1)]
        wspecs += [
            pl.BlockSpec(w_.shape, lambda bi, ti: (0, 0)),
            pl.BlockSpec((1, b_.shape[0]), lambda bi, ti: (0, 0)),
        ]
    cout = mlp[-1][0].shape[1]
    r2 = np.float32(r * r)
    out = pl.pallas_call(
        functools.partial(_sa_kernel, np_=np_, k=k, r2=r2, cdim=cdim,
                          t=t, nw=len(wlist)),
        grid=(b, n_t),
        in_specs=[
            pl.BlockSpec((1, 3, np_), lambda bi, ti: (bi, 0, 0)),
            pl.BlockSpec((1, t, 3), lambda bi, ti: (bi, ti, 0)),
            pl.BlockSpec((1, np_, cdim), lambda bi, ti: (bi, 0, 0)),
        ] + wspecs,
        out_specs=pl.BlockSpec((1, t, cout), lambda bi, ti: (bi, ti, 0)),
        out_shape=jax.ShapeDtypeStruct((b, n_t * t, cout), jnp.float32),
    )(p_t, py_pad, xc, *wlist)
    return out[:, :n_s]


def _fp_kernel(pin_ref, ix_ref, psk_ref, sx_ref, *refs, nin, k, t, nw):
    ws = refs[:nw]
    out_ref = refs[nw]
    pin = pin_ref[0]
    ix = ix_ref[0]
    psk = psk_ref[0]
    sx = sx_ref[0]
    iota = jax.lax.broadcasted_iota(jnp.int32, (1, nin), 1)

    d2 = (psk[:, 0:1] - pin[0:1, :]) ** 2
    d2 = d2 + (psk[:, 1:2] - pin[1:2, :]) ** 2
    d2 = d2 + (psk[:, 2:3] - pin[2:3, :]) ** 2

    idxs = []
    dists = []
    for _ in range(k):
        m = jnp.min(d2, axis=1, keepdims=True)
        idx = jnp.min(jnp.where(d2 == m, iota, nin), axis=1, keepdims=True)
        d2 = jnp.where(iota == idx, _F32MAX, d2)
        idxs.append(idx)
        dists.append(jnp.sqrt(m))

    wsum = jnp.zeros_like(dists[0])
    wts = []
    for d_ in dists:
        w_ = 1.0 / jnp.maximum(d_, np.float32(1e-10))
        wts.append(w_)
        wsum = wsum + w_
    wsum = wsum + np.float32(1e-16)

    a = jnp.zeros((psk.shape[0], nin), dtype=jnp.float32)
    for idx, w_ in zip(idxs, wts):
        a = a + jnp.where(iota == idx, w_ / wsum, 0.0)
    agg = jnp.dot(a, ix, preferred_element_type=jnp.float32)

    h = jnp.concatenate([agg, sx, psk], axis=1)
    for li in range(0, nw, 2):
        h = jax.nn.relu(
            jnp.dot(h, ws[li][...], preferred_element_type=jnp.float32)
            + ws[li + 1][...]
        )
    out_ref[0] = h


def _fp_call(pin_t, ix, psk_n3, sx, mlp, k, t=128):
    b, _, nin = pin_t.shape
    nsk = psk_n3.shape[1]
    c = ix.shape[-1]
    cs = sx.shape[-1]
    n_t = _cdiv(nsk, t)
    pad = n_t * t - nsk
    psk_pad = jnp.pad(psk_n3, ((0, 0), (0, pad), (0, 0)),
                      constant_values=1e9)
    sx_pad = jnp.pad(sx, ((0, 0), (0, pad), (0, 0)))
    wlist = []
    wspecs = []
    for w_, b_ in mlp:
        wlist += [w_, b_.reshape(1, -1)]
        wspecs += [
            pl.BlockSpec(w_.shape, lambda bi, ti: (0, 0)),
            pl.BlockSpec((1, b_.shape[0]), lambda bi, ti: (0, 0)),
        ]
    cout = mlp[-1][0].shape[1]
    out = pl.pallas_call(
        functools.partial(_fp_kernel, nin=nin, k=k, t=t, nw=len(wlist)),
        grid=(b, n_t),
        in_specs=[
            pl.BlockSpec((1, 3, nin), lambda bi, ti: (bi, 0, 0)),
            pl.BlockSpec((1, nin, c), lambda bi, ti: (bi, 0, 0)),
            pl.BlockSpec((1, t, 3), lambda bi, ti: (bi, ti, 0)),
            pl.BlockSpec((1, t, cs), lambda bi, ti: (bi, ti, 0)),
        ] + wspecs,
        out_specs=pl.BlockSpec((1, t, cout), lambda bi, ti: (bi, ti, 0)),
        out_shape=jax.ShapeDtypeStruct((b, n_t * t, cout), jnp.float32),
    )(pin_t, ix, psk_pad, sx_pad, *wlist)
    return out[:, :nsk]


def _clf_kernel(f_ref, *refs, nl, t):
    f = f_ref[0]
    nw = [nl, nl + 1]
    o1_ref, o2_ref = refs[-2], refs[-1]
    offs = 0
    outs = []
    for hd in range(2):
        h = f
        for li in range(nl):
            w_ = refs[offs + li][...]
            h = jnp.dot(h, w_, preferred_element_type=jnp.float32)
            if li == nl - 1:
                h = h + refs[offs + nl][...]
            else:
                h = jax.nn.relu(h)
        outs.append(h)
        offs += nl + 1
    o1_ref[0] = outs[0]
    o2_ref[0] = outs[1]


def _clf_call(f1, clfs, t=512):
    b, n, c = f1.shape
    wlist = []
    wspecs = []
    nl = len(clfs[0])
    for clf in clfs:
        for layer in clf:
            wt = jnp.transpose(layer[0])
            wlist.append(wt)
            wspecs.append(pl.BlockSpec(wt.shape, lambda bi, ti: (0, 0)))
        bias = clf[-1][1].reshape(1, -1)
        wlist.append(bias)
        wspecs.append(pl.BlockSpec(bias.shape, lambda bi, ti: (0, 0)))
    ncls = clfs[0][-1][0].shape[0]
    n_t = _cdiv(n, t)
    o1, o2 = pl.pallas_call(
        functools.partial(_clf_kernel, nl=nl, t=t),
        grid=(b, n_t),
        in_specs=[pl.BlockSpec((1, t, c), lambda bi, ti: (bi, ti, 0))]
        + wspecs,
        out_specs=[
            pl.BlockSpec((1, t, ncls), lambda bi, ti: (bi, ti, 0)),
            pl.BlockSpec((1, t, ncls), lambda bi, ti: (bi, ti, 0)),
        ],
        out_shape=[
            jax.ShapeDtypeStruct((b, n, ncls), jnp.float32),
            jax.ShapeDtypeStruct((b, n, ncls), jnp.float32),
        ],
    )(f1, *wlist)
    return o1, o2


def kernel(data, params):
    pos = jnp.transpose(data, (0, 2, 1))
    pos_t = data
    x = pos

    sa_cfg = [
        ("sa1", 0.8, 0.025, 64),
        ("sa2", 0.7, 0.05, 64),
        ("sa3", 0.6, 0.1, 64),
        ("sa4", 0.5, 0.2, 64),
    ]
    p_t = pos_t
    p_n3 = pos
    xs = [x]
    ps_t = [p_t]
    ps_n3 = [p_n3]
    cur_x = x
    for name, ratio, r, k in sa_cfg:
        np_ = p_t.shape[2]
        n_s = int(math.ceil(ratio * np_))
        py_t = _fps_call(p_t, n_s)
        py_n3 = jnp.transpose(py_t, (0, 2, 1))
        cur_x = _sa_call(p_t, py_n3, cur_x, params[name], r, k)
        p_t, p_n3 = py_t, py_n3
        xs.append(cur_x)
        ps_t.append(p_t)
        ps_n3.append(p_n3)

    f = xs[4]
    fp_cfg = [("fp4", 1, 3), ("fp3", 1, 2), ("fp2", 3, 1), ("fp1", 3, 0)]
    for name, k, si in fp_cfg:
        f = _fp_call(ps_t[si + 1], f, ps_n3[si], xs[si], params[name], k)

    o1, o2 = _clf_call(f, params["classifiers"])
    return (jnp.transpose(o1, (0, 2, 1)), jnp.transpose(o2, (0, 2, 1)))


# speedup vs baseline: 7.7716x; 1.0001x over previous
import functools
import math

import jax
import jax.numpy as jnp
import numpy as np
from jax.experimental import pallas as pl
from jax.experimental.pallas import tpu as pltpu

_NEG = np.float32(-1e30)
_F32MAX = np.float32(3e38)


def _cdiv(a, b):
    return (a + b - 1) // b


def _fps_kernel(p_ref, py_ref, *, n_s, npp):
    s = npp // 128
    p = p_ref[0].reshape(3, s, 128)
    il = (jax.lax.broadcasted_iota(jnp.int32, (s, 128), 0) * 128
          + jax.lax.broadcasted_iota(jnp.int32, (s, 128), 1))
    ic = jax.lax.broadcasted_iota(jnp.int32, (1, 128), 1)

    def step(j, st):
        dist, cur, cx, cy, cz = st
        m = (il == cur).astype(jnp.float32)
        px = jnp.sum(p[0] * m)
        py_ = jnp.sum(p[1] * m)
        pz = jnp.sum(p[2] * m)
        cx = jnp.where(ic == j, px, cx)
        cy = jnp.where(ic == j, py_, cy)
        cz = jnp.where(ic == j, pz, cz)
        d = (p[0] - px) ** 2 + (p[1] - py_) ** 2 + (p[2] - pz) ** 2
        dist = jnp.minimum(dist, d)
        mx = jnp.max(dist)
        cur = jnp.min(jnp.where(dist == mx, il, npp))
        return dist, cur, cx, cy, cz

    dist = jnp.full((s, 128), _F32MAX, dtype=jnp.float32)
    cur = jnp.int32(0)
    z = jnp.zeros((1, 128), dtype=jnp.float32)
    for c0 in range(0, n_s, 128):
        w = min(128, n_s - c0)
        dist, cur, cx, cy, cz = jax.lax.fori_loop(
            0, w, step, (dist, cur, z, z, z))
        py_ref[0, 0:1, c0:c0 + w] = cx[:, :w]
        py_ref[0, 1:2, c0:c0 + w] = cy[:, :w]
        py_ref[0, 2:3, c0:c0 + w] = cz[:, :w]


def _fps_call(p_t, n_s):
    b, _, np_ = p_t.shape
    npp = _cdiv(np_, 128) * 128
    if npp > np_:
        p_t = jnp.concatenate(
            [p_t, jnp.broadcast_to(p_t[:, :, :1], (b, 3, npp - np_))], axis=2)
    return pl.pallas_call(
        functools.partial(_fps_kernel, n_s=n_s, npp=npp),
        grid=(b,),
        compiler_params=pltpu.CompilerParams(
            dimension_semantics=("parallel",)),
        in_specs=[pl.BlockSpec((1, 3, npp), lambda i: (i, 0, 0))],
        out_specs=pl.BlockSpec((1, 3, n_s), lambda i: (i, 0, 0)),
        out_shape=jax.ShapeDtypeStruct((b, 3, n_s), jnp.float32),
    )(p_t)


def _sa_kernel(p_ref, py_ref, xc_ref, *refs, np_, k, r2, cdim, t, nw):
    ws = refs[:nw]
    out_ref = refs[nw]
    p = p_ref[0]
    py = py_ref[0]
    xc = xc_ref[0]
    iota = jax.lax.broadcasted_iota(jnp.int32, (1, np_), 1)

    d2 = (py[:, 0:1] - p[0:1, :]) ** 2
    d2 = d2 + (py[:, 1:2] - p[1:2, :]) ** 2
    d2 = d2 + (py[:, 2:3] - p[2:3, :]) ** 2
    valid0 = d2 <= r2
    order = jnp.where(valid0, iota, np_)
    cnt = jnp.sum(valid0.astype(jnp.int32), axis=1, keepdims=True)
    kmax = jnp.minimum(jnp.max(cnt), k)

    cout = ws[nw - 2].shape[1]

    def body(st):
        ki, order, acc = st
        mk = jnp.min(order, axis=1, keepdims=True)
        order = jnp.where(order == mk, np_ + 1, order)
        valid = mk < np_
        oh = (iota == mk).astype(jnp.float32)
        gj = jnp.dot(oh, xc, preferred_element_type=jnp.float32)
        dp = gj[:, cdim - 3:] - py
        h = jnp.concatenate([gj[:, : cdim - 3], dp], axis=1)
        for li in range(0, nw, 2):
            h = jax.nn.relu(
                jnp.dot(h, ws[li][...], preferred_element_type=jnp.float32)
                + ws[li + 1][...]
            )
        acc = jnp.maximum(acc, jnp.where(valid, h, _NEG))
        return ki + 1, order, acc

    acc0 = jnp.full((t, cout), _NEG, dtype=jnp.float32)
    _, _, acc = jax.lax.while_loop(
        lambda st: st[0] < kmax, body, (jnp.int32(0), order, acc0))
    out_ref[0] = acc


def _sa_call(p_t, py_n3, x, mlp, r, k, t=128):
    b, _, np_ = p_t.shape
    n_s = py_n3.shape[1]
    cdim = x.shape[-1] + 3
    xc = jnp.concatenate([x, jnp.transpose(p_t, (0, 2, 1))], axis=-1)
    n_t = _cdiv(n_s, t)
    pad = n_t * t - n_s
    py_pad = jnp.pad(py_n3, ((0, 0), (0, pad), (0, 0)),
                     constant_values=1e9)
    wlist = []
    wspecs = []
    for w_, b_ in mlp:
        wlist += [w_, b_.reshape(1, -1)]
        wspecs += [
            pl.BlockSpec(w_.shape, lambda bi, ti: (0, 0)),
            pl.BlockSpec((1, b_.shape[0]), lambda bi, ti: (0, 0)),
        ]
    cout = mlp[-1][0].shape[1]
    r2 = np.float32(r * r)
    out = pl.pallas_call(
        functools.partial(_sa_kernel, np_=np_, k=k, r2=r2, cdim=cdim,
                          t=t, nw=len(wlist)),
        grid=(b, n_t),
        compiler_params=pltpu.CompilerParams(
            dimension_semantics=("parallel", "parallel")),
        in_specs=[
            pl.BlockSpec((1, 3, np_), lambda bi, ti: (bi, 0, 0)),
            pl.BlockSpec((1, t, 3), lambda bi, ti: (bi, ti, 0)),
            pl.BlockSpec((1, np_, cdim), lambda bi, ti: (bi, 0, 0)),
        ] + wspecs,
        out_specs=pl.BlockSpec((1, t, cout), lambda bi, ti: (bi, ti, 0)),
        out_shape=jax.ShapeDtypeStruct((b, n_t * t, cout), jnp.float32),
    )(p_t, py_pad, xc, *wlist)
    return out[:, :n_s]


def _fp_kernel(pin_ref, ix_ref, psk_ref, sx_ref, *refs, nin, k, t, nw):
    ws = refs[:nw]
    out_ref = refs[nw]
    pin = pin_ref[0]
    ix = ix_ref[0]
    psk = psk_ref[0]
    sx = sx_ref[0]
    iota = jax.lax.broadcasted_iota(jnp.int32, (1, nin), 1)

    d2 = (psk[:, 0:1] - pin[0:1, :]) ** 2
    d2 = d2 + (psk[:, 1:2] - pin[1:2, :]) ** 2
    d2 = d2 + (psk[:, 2:3] - pin[2:3, :]) ** 2

    idxs = []
    dists = []
    for _ in range(k):
        m = jnp.min(d2, axis=1, keepdims=True)
        idx = jnp.min(jnp.where(d2 == m, iota, nin), axis=1, keepdims=True)
        d2 = jnp.where(iota == idx, _F32MAX, d2)
        idxs.append(idx)
        dists.append(jnp.sqrt(m))

    wsum = jnp.zeros_like(dists[0])
    wts = []
    for d_ in dists:
        w_ = 1.0 / jnp.maximum(d_, np.float32(1e-10))
        wts.append(w_)
        wsum = wsum + w_
    wsum = wsum + np.float32(1e-16)

    a = jnp.zeros((psk.shape[0], nin), dtype=jnp.float32)
    for idx, w_ in zip(idxs, wts):
        a = a + jnp.where(iota == idx, w_ / wsum, 0.0)
    agg = jnp.dot(a, ix, preferred_element_type=jnp.float32)

    h = jnp.concatenate([agg, sx, psk], axis=1)
    for li in range(0, nw, 2):
        h = jax.nn.relu(
            jnp.dot(h, ws[li][...], preferred_element_type=jnp.float32)
            + ws[li + 1][...]
        )
    out_ref[0] = h


def _fp_call(pin_t, ix, psk_n3, sx, mlp, k, t=128):
    b, _, nin = pin_t.shape
    nsk = psk_n3.shape[1]
    c = ix.shape[-1]
    cs = sx.shape[-1]
    n_t = _cdiv(nsk, t)
    pad = n_t * t - nsk
    psk_pad = jnp.pad(psk_n3, ((0, 0), (0, pad), (0, 0)),
                      constant_values=1e9)
    sx_pad = jnp.pad(sx, ((0, 0), (0, pad), (0, 0)))
    wlist = []
    wspecs = []
    for w_, b_ in mlp:
        wlist += [w_, b_.reshape(1, -1)]
        wspecs += [
            pl.BlockSpec(w_.shape, lambda bi, ti: (0, 0)),
            pl.BlockSpec((1, b_.shape[0]), lambda bi, ti: (0, 0)),
        ]
    cout = mlp[-1][0].shape[1]
    out = pl.pallas_call(
        functools.partial(_fp_kernel, nin=nin, k=k, t=t, nw=len(wlist)),
        grid=(b, n_t),
        compiler_params=pltpu.CompilerParams(
            dimension_semantics=("parallel", "parallel")),
        in_specs=[
            pl.BlockSpec((1, 3, nin), lambda bi, ti: (bi, 0, 0)),
            pl.BlockSpec((1, nin, c), lambda bi, ti: (bi, 0, 0)),
            pl.BlockSpec((1, t, 3), lambda bi, ti: (bi, ti, 0)),
            pl.BlockSpec((1, t, cs), lambda bi, ti: (bi, ti, 0)),
        ] + wspecs,
        out_specs=pl.BlockSpec((1, t, cout), lambda bi, ti: (bi, ti, 0)),
        out_shape=jax.ShapeDtypeStruct((b, n_t * t, cout), jnp.float32),
    )(pin_t, ix, psk_pad, sx_pad, *wlist)
    return out[:, :nsk]


def _clf_kernel(f_ref, *refs, nl, t):
    f = f_ref[0]
    nw = [nl, nl + 1]
    o1_ref, o2_ref = refs[-2], refs[-1]
    offs = 0
    outs = []
    for hd in range(2):
        h = f
        for li in range(nl):
            w_ = refs[offs + li][...]
            h = jnp.dot(h, w_, preferred_element_type=jnp.float32)
            if li == nl - 1:
                h = h + refs[offs + nl][...]
            else:
                h = jax.nn.relu(h)
        outs.append(h)
        offs += nl + 1
    o1_ref[0] = outs[0]
    o2_ref[0] = outs[1]


def _clf_call(f1, clfs, t=512):
    b, n, c = f1.shape
    wlist = []
    wspecs = []
    nl = len(clfs[0])
    for clf in clfs:
        for layer in clf:
            wt = jnp.transpose(layer[0])
            wlist.append(wt)
            wspecs.append(pl.BlockSpec(wt.shape, lambda bi, ti: (0, 0)))
        bias = clf[-1][1].reshape(1, -1)
        wlist.append(bias)
        wspecs.append(pl.BlockSpec(bias.shape, lambda bi, ti: (0, 0)))
    ncls = clfs[0][-1][0].shape[0]
    n_t = _cdiv(n, t)
    o1, o2 = pl.pallas_call(
        functools.partial(_clf_kernel, nl=nl, t=t),
        grid=(b, n_t),
        compiler_params=pltpu.CompilerParams(
            dimension_semantics=("parallel", "parallel")),
        in_specs=[pl.BlockSpec((1, t, c), lambda bi, ti: (bi, ti, 0))]
        + wspecs,
        out_specs=[
            pl.BlockSpec((1, t, ncls), lambda bi, ti: (bi, ti, 0)),
            pl.BlockSpec((1, t, ncls), lambda bi, ti: (bi, ti, 0)),
        ],
        out_shape=[
            jax.ShapeDtypeStruct((b, n, ncls), jnp.float32),
            jax.ShapeDtypeStruct((b, n, ncls), jnp.float32),
        ],
    )(f1, *wlist)
    return o1, o2


def kernel(data, params):
    pos = jnp.transpose(data, (0, 2, 1))
    pos_t = data
    x = pos

    sa_cfg = [
        ("sa1", 0.8, 0.025, 64),
        ("sa2", 0.7, 0.05, 64),
        ("sa3", 0.6, 0.1, 64),
        ("sa4", 0.5, 0.2, 64),
    ]
    p_t = pos_t
    p_n3 = pos
    xs = [x]
    ps_t = [p_t]
    ps_n3 = [p_n3]
    cur_x = x
    for name, ratio, r, k in sa_cfg:
        np_ = p_t.shape[2]
        n_s = int(math.ceil(ratio * np_))
        py_t = _fps_call(p_t, n_s)
        py_n3 = jnp.transpose(py_t, (0, 2, 1))
        cur_x = _sa_call(p_t, py_n3, cur_x, params[name], r, k)
        p_t, p_n3 = py_t, py_n3
        xs.append(cur_x)
        ps_t.append(p_t)
        ps_n3.append(p_n3)

    f = xs[4]
    fp_cfg = [("fp4", 1, 3), ("fp3", 1, 2), ("fp2", 3, 1), ("fp1", 3, 0)]
    for name, k, si in fp_cfg:
        f = _fp_call(ps_t[si + 1], f, ps_n3[si], xs[si], params[name], k)

    o1, o2 = _clf_call(f, params["classifiers"])
    return (jnp.transpose(o1, (0, 2, 1)), jnp.transpose(o2, (0, 2, 1)))


# speedup vs baseline: 9.7132x; 1.2498x over previous
import functools
import math

import jax
import jax.numpy as jnp
import numpy as np
from jax.experimental import pallas as pl
from jax.experimental.pallas import tpu as pltpu

_NEG = np.float32(-1e30)
_F32MAX = np.float32(3e38)


def _cdiv(a, b):
    return (a + b - 1) // b


def _fps_kernel(pt_ref, pn_ref, py_ref, d2_ref, *, n_s, npp):
    pn = pn_ref[0]
    p2 = pt_ref[0]
    for c in range(3):
        sq = (pn[:, c:c + 1] - p2[c:c + 1, :]) ** 2
        if c == 0:
            d2_ref[...] = sq
        else:
            d2_ref[...] += sq

    ir = jax.lax.broadcasted_iota(jnp.int32, (1, npp), 1)
    icol = jax.lax.broadcasted_iota(jnp.int32, (128, 1), 0)

    def step(j, st):
        dist, cur, selc = st
        selc = jnp.where(icol == j, cur, selc)
        dist = jnp.minimum(dist, d2_ref[pl.ds(cur, 1), :])
        mx = jnp.max(dist)
        cur = jnp.min(jnp.where(dist == mx, ir, npp))
        return dist, cur, selc

    dist = jnp.full((1, npp), _F32MAX, dtype=jnp.float32)
    cur = jnp.int32(0)
    z = jnp.zeros((128, 1), dtype=jnp.int32)
    for c0 in range(0, n_s, 128):
        w = min(128, n_s - c0)
        dist, cur, selc = jax.lax.fori_loop(0, w, step, (dist, cur, z))
        oh = (selc == ir).astype(jnp.float32)
        pyc = jnp.dot(oh, pn, preferred_element_type=jnp.float32)
        py_ref[0, c0:c0 + w, :] = pyc[:w]


def _fps_call(p_t, p_n3, n_s):
    b, _, np_ = p_t.shape
    npp = _cdiv(np_, 128) * 128
    if npp > np_:
        p_t = jnp.concatenate(
            [p_t, jnp.broadcast_to(p_t[:, :, :1], (b, 3, npp - np_))], axis=2)
        p_n3 = jnp.concatenate(
            [p_n3, jnp.broadcast_to(p_n3[:, :1, :], (b, npp - np_, 3))],
            axis=1)
    return pl.pallas_call(
        functools.partial(_fps_kernel, n_s=n_s, npp=npp),
        grid=(b,),
        compiler_params=pltpu.CompilerParams(
            dimension_semantics=("parallel",)),
        in_specs=[
            pl.BlockSpec((1, 3, npp), lambda i: (i, 0, 0)),
            pl.BlockSpec((1, npp, 3), lambda i: (i, 0, 0)),
        ],
        out_specs=pl.BlockSpec((1, n_s, 3), lambda i: (i, 0, 0)),
        out_shape=jax.ShapeDtypeStruct((b, n_s, 3), jnp.float32),
        scratch_shapes=[pltpu.VMEM((npp, npp), jnp.float32)],
    )(p_t, p_n3)


def _sa_kernel(p_ref, py_ref, xc_ref, *refs, np_, k, r2, cdim, t, nw):
    ws = refs[:nw]
    out_ref = refs[nw]
    p = p_ref[0]
    py = py_ref[0]
    xc = xc_ref[0]
    iota = jax.lax.broadcasted_iota(jnp.int32, (1, np_), 1)

    d2 = (py[:, 0:1] - p[0:1, :]) ** 2
    d2 = d2 + (py[:, 1:2] - p[1:2, :]) ** 2
    d2 = d2 + (py[:, 2:3] - p[2:3, :]) ** 2
    valid0 = d2 <= r2
    order = jnp.where(valid0, iota, np_)
    cnt = jnp.sum(valid0.astype(jnp.int32), axis=1, keepdims=True)
    kmax = jnp.minimum(jnp.max(cnt), k)

    cout = ws[nw - 2].shape[1]

    def body(st):
        ki, order, acc = st
        mk = jnp.min(order, axis=1, keepdims=True)
        order = jnp.where(order == mk, np_ + 1, order)
        valid = mk < np_
        oh = (iota == mk).astype(jnp.float32)
        gj = jnp.dot(oh, xc, preferred_element_type=jnp.float32)
        dp = gj[:, cdim - 3:] - py
        h = jnp.concatenate([gj[:, : cdim - 3], dp], axis=1)
        for li in range(0, nw, 2):
            h = jax.nn.relu(
                jnp.dot(h, ws[li][...], preferred_element_type=jnp.float32)
                + ws[li + 1][...]
            )
        acc = jnp.maximum(acc, jnp.where(valid, h, _NEG))
        return ki + 1, order, acc

    acc0 = jnp.full((t, cout), _NEG, dtype=jnp.float32)
    _, _, acc = jax.lax.while_loop(
        lambda st: st[0] < kmax, body, (jnp.int32(0), order, acc0))
    out_ref[0] = acc


def _sa_call(p_t, py_n3, x, mlp, r, k, t=128):
    b, _, np_ = p_t.shape
    n_s = py_n3.shape[1]
    cdim = x.shape[-1] + 3
    xc = jnp.concatenate([x, jnp.transpose(p_t, (0, 2, 1))], axis=-1)
    n_t = _cdiv(n_s, t)
    pad = n_t * t - n_s
    py_pad = jnp.pad(py_n3, ((0, 0), (0, pad), (0, 0)),
                     constant_values=1e9)
    wlist = []
    wspecs = []
    for w_, b_ in mlp:
        wlist += [w_, b_.reshape(1, -1)]
        wspecs += [
            pl.BlockSpec(w_.shape, lambda bi, ti: (0, 0)),
            pl.BlockSpec((1, b_.shape[0]), lambda bi, ti: (0, 0)),
        ]
    cout = mlp[-1][0].shape[1]
    r2 = np.float32(r * r)
    out = pl.pallas_call(
        functools.partial(_sa_kernel, np_=np_, k=k, r2=r2, cdim=cdim,
                          t=t, nw=len(wlist)),
        grid=(b, n_t),
        compiler_params=pltpu.CompilerParams(
            dimension_semantics=("parallel", "parallel")),
        in_specs=[
            pl.BlockSpec((1, 3, np_), lambda bi, ti: (bi, 0, 0)),
            pl.BlockSpec((1, t, 3), lambda bi, ti: (bi, ti, 0)),
            pl.BlockSpec((1, np_, cdim), lambda bi, ti: (bi, 0, 0)),
        ] + wspecs,
        out_specs=pl.BlockSpec((1, t, cout), lambda bi, ti: (bi, ti, 0)),
        out_shape=jax.ShapeDtypeStruct((b, n_t * t, cout), jnp.float32),
    )(p_t, py_pad, xc, *wlist)
    return out[:, :n_s]


def _fp_kernel(pin_ref, ix_ref, psk_ref, sx_ref, *refs, nin, k, t, nw):
    ws = refs[:nw]
    out_ref = refs[nw]
    pin = pin_ref[0]
    ix = ix_ref[0]
    psk = psk_ref[0]
    sx = sx_ref[0]
    iota = jax.lax.broadcasted_iota(jnp.int32, (1, nin), 1)

    d2 = (psk[:, 0:1] - pin[0:1, :]) ** 2
    d2 = d2 + (psk[:, 1:2] - pin[1:2, :]) ** 2
    d2 = d2 + (psk[:, 2:3] - pin[2:3, :]) ** 2

    idxs = []
    dists = []
    for _ in range(k):
        m = jnp.min(d2, axis=1, keepdims=True)
        idx = jnp.min(jnp.where(d2 == m, iota, nin), axis=1, keepdims=True)
        d2 = jnp.where(iota == idx, _F32MAX, d2)
        idxs.append(idx)
        dists.append(jnp.sqrt(m))

    wsum = jnp.zeros_like(dists[0])
    wts = []
    for d_ in dists:
        w_ = 1.0 / jnp.maximum(d_, np.float32(1e-10))
        wts.append(w_)
        wsum = wsum + w_
    wsum = wsum + np.float32(1e-16)

    a = jnp.zeros((psk.shape[0], nin), dtype=jnp.float32)
    for idx, w_ in zip(idxs, wts):
        a = a + jnp.where(iota == idx, w_ / wsum, 0.0)
    agg = jnp.dot(a, ix, preferred_element_type=jnp.float32)

    h = jnp.concatenate([agg, sx, psk], axis=1)
    for li in range(0, nw, 2):
        h = jax.nn.relu(
            jnp.dot(h, ws[li][...], preferred_element_type=jnp.float32)
            + ws[li + 1][...]
        )
    out_ref[0] = h


def _fp_call(pin_t, ix, psk_n3, sx, mlp, k, t=128):
    b, _, nin = pin_t.shape
    nsk = psk_n3.shape[1]
    c = ix.shape[-1]
    cs = sx.shape[-1]
    n_t = _cdiv(nsk, t)
    pad = n_t * t - nsk
    psk_pad = jnp.pad(psk_n3, ((0, 0), (0, pad), (0, 0)),
                      constant_values=1e9)
    sx_pad = jnp.pad(sx, ((0, 0), (0, pad), (0, 0)))
    wlist = []
    wspecs = []
    for w_, b_ in mlp:
        wlist += [w_, b_.reshape(1, -1)]
        wspecs += [
            pl.BlockSpec(w_.shape, lambda bi, ti: (0, 0)),
            pl.BlockSpec((1, b_.shape[0]), lambda bi, ti: (0, 0)),
        ]
    cout = mlp[-1][0].shape[1]
    out = pl.pallas_call(
        functools.partial(_fp_kernel, nin=nin, k=k, t=t, nw=len(wlist)),
        grid=(b, n_t),
        compiler_params=pltpu.CompilerParams(
            dimension_semantics=("parallel", "parallel")),
        in_specs=[
            pl.BlockSpec((1, 3, nin), lambda bi, ti: (bi, 0, 0)),
            pl.BlockSpec((1, nin, c), lambda bi, ti: (bi, 0, 0)),
            pl.BlockSpec((1, t, 3), lambda bi, ti: (bi, ti, 0)),
            pl.BlockSpec((1, t, cs), lambda bi, ti: (bi, ti, 0)),
        ] + wspecs,
        out_specs=pl.BlockSpec((1, t, cout), lambda bi, ti: (bi, ti, 0)),
        out_shape=jax.ShapeDtypeStruct((b, n_t * t, cout), jnp.float32),
    )(pin_t, ix, psk_pad, sx_pad, *wlist)
    return out[:, :nsk]


def _clf_kernel(f_ref, *refs, nl, t):
    f = f_ref[0]
    nw = [nl, nl + 1]
    o1_ref, o2_ref = refs[-2], refs[-1]
    offs = 0
    outs = []
    for hd in range(2):
        h = f
        for li in range(nl):
            w_ = refs[offs + li][...]
            h = jnp.dot(h, w_, preferred_element_type=jnp.float32)
            if li == nl - 1:
                h = h + refs[offs + nl][...]
            else:
                h = jax.nn.relu(h)
        outs.append(h)
        offs += nl + 1
    o1_ref[0] = outs[0]
    o2_ref[0] = outs[1]


def _clf_call(f1, clfs, t=512):
    b, n, c = f1.shape
    wlist = []
    wspecs = []
    nl = len(clfs[0])
    for clf in clfs:
        for layer in clf:
            wt = jnp.transpose(layer[0])
            wlist.append(wt)
            wspecs.append(pl.BlockSpec(wt.shape, lambda bi, ti: (0, 0)))
        bias = clf[-1][1].reshape(1, -1)
        wlist.append(bias)
        wspecs.append(pl.BlockSpec(bias.shape, lambda bi, ti: (0, 0)))
    ncls = clfs[0][-1][0].shape[0]
    n_t = _cdiv(n, t)
    o1, o2 = pl.pallas_call(
        functools.partial(_clf_kernel, nl=nl, t=t),
        grid=(b, n_t),
        compiler_params=pltpu.CompilerParams(
            dimension_semantics=("parallel", "parallel")),
        in_specs=[pl.BlockSpec((1, t, c), lambda bi, ti: (bi, ti, 0))]
        + wspecs,
        out_specs=[
            pl.BlockSpec((1, t, ncls), lambda bi, ti: (bi, ti, 0)),
            pl.BlockSpec((1, t, ncls), lambda bi, ti: (bi, ti, 0)),
        ],
        out_shape=[
            jax.ShapeDtypeStruct((b, n, ncls), jnp.float32),
            jax.ShapeDtypeStruct((b, n, ncls), jnp.float32),
        ],
    )(f1, *wlist)
    return o1, o2


def kernel(data, params):
    pos = jnp.transpose(data, (0, 2, 1))
    pos_t = data
    x = pos

    sa_cfg = [
        ("sa1", 0.8, 0.025, 64),
        ("sa2", 0.7, 0.05, 64),
        ("sa3", 0.6, 0.1, 64),
        ("sa4", 0.5, 0.2, 64),
    ]
    p_t = pos_t
    p_n3 = pos
    xs = [x]
    ps_t = [p_t]
    ps_n3 = [p_n3]
    cur_x = x
    for name, ratio, r, k in sa_cfg:
        np_ = p_t.shape[2]
        n_s = int(math.ceil(ratio * np_))
        py_n3 = _fps_call(p_t, p_n3, n_s)
        py_t = jnp.transpose(py_n3, (0, 2, 1))
        cur_x = _sa_call(p_t, py_n3, cur_x, params[name], r, k)
        p_t, p_n3 = py_t, py_n3
        xs.append(cur_x)
        ps_t.append(p_t)
        ps_n3.append(p_n3)

    f = xs[4]
    fp_cfg = [("fp4", 1, 3), ("fp3", 1, 2), ("fp2", 3, 1), ("fp1", 3, 0)]
    for name, k, si in fp_cfg:
        f = _fp_call(ps_t[si + 1], f, ps_n3[si], xs[si], params[name], k)

    o1, o2 = _clf_call(f, params["classifiers"])
    return (jnp.transpose(o1, (0, 2, 1)), jnp.transpose(o2, (0, 2, 1)))
